# baseline (speedup 1.0000x reference)
import jax
import jax.numpy as jnp
from jax.experimental import pallas as pl
from jax.experimental.pallas import tpu as pltpu

_N0, _N1, _N2, _D, _G = 2048, 4096, 512, 32, 8
_HI = jax.lax.Precision.HIGHEST


def _dot(a, b):
    return jax.lax.dot_general(a, b, (((1,), (0,)), ((), ())),
                               precision=_HI, preferred_element_type=jnp.float32)


def _dott(a, b):
    return jax.lax.dot_general(a, b, (((0,), (0,)), ((), ())),
                               precision=_HI, preferred_element_type=jnp.float32)


def _mm_body(a_ref, x_ref, o_ref):
    o_ref[...] = _dot(a_ref[...], x_ref[...])


def _mm(A, X, bm):
    M, K = A.shape
    Dx = X.shape[1]
    return pl.pallas_call(
        _mm_body,
        grid=(M // bm,),
        in_specs=[pl.BlockSpec((bm, K), lambda i: (i, 0)),
                  pl.BlockSpec((K, Dx), lambda i: (0, 0))],
        out_specs=pl.BlockSpec((bm, Dx), lambda i: (i, 0)),
        out_shape=jax.ShapeDtypeStruct((M, Dx), jnp.float32),
        compiler_params=pltpu.CompilerParams(dimension_semantics=("parallel",)),
    )(A, X)


def _mmt_body(a_ref, x_ref, o_ref):
    o_ref[...] = _dott(a_ref[...], x_ref[...])


def _mmt(A, X, bn):
    M, N = A.shape
    Dx = X.shape[1]
    return pl.pallas_call(
        _mmt_body,
        grid=(N // bn,),
        in_specs=[pl.BlockSpec((M, bn), lambda j: (0, j)),
                  pl.BlockSpec((M, Dx), lambda j: (0, 0))],
        out_specs=pl.BlockSpec((bn, Dx), lambda j: (j, 0)),
        out_shape=jax.ShapeDtypeStruct((N, Dx), jnp.float32),
        compiler_params=pltpu.CompilerParams(dimension_semantics=("parallel",)),
    )(A, X)


def _a0_body(a_ref, w_ref, add_ref, h0_ref, u_ref):
    y = _dott(a_ref[...], w_ref[...])
    h0_ref[...] = y[:, :_D] + add_ref[...]
    u_ref[...] = y[:, _D:]


def _a0_pass(A0, W, add, bn):
    M, N = A0.shape
    Dw = W.shape[1]
    return pl.pallas_call(
        _a0_body,
        grid=(N // bn,),
        in_specs=[pl.BlockSpec((M, bn), lambda j: (0, j)),
                  pl.BlockSpec((M, Dw), lambda j: (0, 0)),
                  pl.BlockSpec((bn, _D), lambda j: (j, 0))],
        out_specs=[pl.BlockSpec((bn, _D), lambda j: (j, 0)),
                   pl.BlockSpec((bn, _D), lambda j: (j, 0))],
        out_shape=[jax.ShapeDtypeStruct((N, _D), jnp.float32),
                   jax.ShapeDtypeStruct((N, _D), jnp.float32)],
        compiler_params=pltpu.CompilerParams(dimension_semantics=("parallel",)),
    )(A0, W, add)


def _a1_body(a_ref, c_ref, e_ref, p_ref, o_ref):
    o_ref[...] = _dott(a_ref[...] + c_ref[...], e_ref[...]) + p_ref[...]


def _a1_pass(A1, coA1, edge, p, bn):
    M, N = A1.shape
    return pl.pallas_call(
        _a1_body,
        grid=(N // bn,),
        in_specs=[pl.BlockSpec((M, bn), lambda j: (0, j)),
                  pl.BlockSpec((M, bn), lambda j: (0, j)),
                  pl.BlockSpec((M, _D), lambda j: (0, 0)),
                  pl.BlockSpec((bn, _D), lambda j: (j, 0))],
        out_specs=pl.BlockSpec((bn, _D), lambda j: (j, 0)),
        out_shape=jax.ShapeDtypeStruct((N, _D), jnp.float32),
        compiler_params=pltpu.CompilerParams(dimension_semantics=("parallel",)),
    )(A1, coA1, edge, p)


def _pool_body(idx_ref, h0_ref, o_ref):
    idx = idx_ref[0, :]
    onehot = (jax.lax.broadcasted_iota(jnp.int32, (_G, _N0), 0)
              == idx[None, :]).astype(jnp.float32)
    s = _dot(onehot, h0_ref[...])
    cnt = jnp.sum(onehot, axis=1, keepdims=True)
    o_ref[...] = s / jnp.maximum(cnt, 1.0)


def _pool(batch_idx2d, h0):
    return pl.pallas_call(
        _pool_body,
        in_specs=[pl.BlockSpec((1, _N0), lambda: (0, 0)),
                  pl.BlockSpec((_N0, _D), lambda: (0, 0))],
        out_specs=pl.BlockSpec((_G, _D), lambda: (0, 0)),
        out_shape=jax.ShapeDtypeStruct((_G, _D), jnp.float32),
    )(batch_idx2d, h0)


def kernel(x, edge_attr, sse_attr, B0, B1, A0, A1, coA1, batch_idx):
    t1h = _mm(B1, sse_attr * 0.5, bm=1024)
    y2 = _mm(B0, jnp.concatenate([t1h, edge_attr], axis=1), bm=256)
    t2, b0e = y2[:, :_D], y2[:, _D:]
    h0, u = _a0_pass(A0, jnp.concatenate([x, t2], axis=1), t2 + b0e, bn=256)
    y4 = _mmt(B0, jnp.concatenate([t2 + x, 0.5 * (u + x)], axis=1), bn=256)
    p, q = y4[:, :_D], y4[:, _D:]
    h1 = _a1_pass(A1, coA1, edge_attr, p, bn=256)
    h2 = _mmt(B1, q + edge_attr, bn=128)
    graph_emb = _pool(batch_idx.reshape(1, _N0).astype(jnp.int32), h0)
    return h0, h1, h2, graph_emb


# speedup vs baseline: 1.5426x; 1.5426x over previous
import jax
import jax.numpy as jnp
from jax.experimental import pallas as pl
from jax.experimental.pallas import tpu as pltpu

_N0, _N1, _N2, _D, _G = 2048, 4096, 512, 32, 8
_HI = jax.lax.Precision.DEFAULT


def _dot(a, b):
    return jax.lax.dot_general(a, b, (((1,), (0,)), ((), ())),
                               precision=_HI, preferred_element_type=jnp.float32)


def _dott(a, b):
    return jax.lax.dot_general(a, b, (((0,), (0,)), ((), ())),
                               precision=_HI, preferred_element_type=jnp.float32)


def _mm_body(a_ref, x_ref, o_ref):
    o_ref[...] = _dot(a_ref[...], x_ref[...])


def _mm(A, X, bm):
    M, K = A.shape
    Dx = X.shape[1]
    return pl.pallas_call(
        _mm_body,
        grid=(M // bm,),
        in_specs=[pl.BlockSpec((bm, K), lambda i: (i, 0)),
                  pl.BlockSpec((K, Dx), lambda i: (0, 0))],
        out_specs=pl.BlockSpec((bm, Dx), lambda i: (i, 0)),
        out_shape=jax.ShapeDtypeStruct((M, Dx), jnp.float32),
        compiler_params=pltpu.CompilerParams(dimension_semantics=("parallel",)),
    )(A, X)


def _mmt_body(a_ref, x_ref, o_ref):
    o_ref[...] = _dott(a_ref[...], x_ref[...])


def _mmt(A, X, bn):
    M, N = A.shape
    Dx = X.shape[1]
    return pl.pallas_call(
        _mmt_body,
        grid=(N // bn,),
        in_specs=[pl.BlockSpec((M, bn), lambda j: (0, j)),
                  pl.BlockSpec((M, Dx), lambda j: (0, 0))],
        out_specs=pl.BlockSpec((bn, Dx), lambda j: (j, 0)),
        out_shape=jax.ShapeDtypeStruct((N, Dx), jnp.float32),
        compiler_params=pltpu.CompilerParams(dimension_semantics=("parallel",)),
    )(A, X)


def _a0_body(a_ref, w_ref, add_ref, h0_ref, u_ref):
    y = _dott(a_ref[...], w_ref[...])
    h0_ref[...] = y[:, :_D] + add_ref[...]
    u_ref[...] = y[:, _D:]


def _a0_pass(A0, W, add, bn):
    M, N = A0.shape
    Dw = W.shape[1]
    return pl.pallas_call(
        _a0_body,
        grid=(N // bn,),
        in_specs=[pl.BlockSpec((M, bn), lambda j: (0, j)),
                  pl.BlockSpec((M, Dw), lambda j: (0, 0)),
                  pl.BlockSpec((bn, _D), lambda j: (j, 0))],
        out_specs=[pl.BlockSpec((bn, _D), lambda j: (j, 0)),
                   pl.BlockSpec((bn, _D), lambda j: (j, 0))],
        out_shape=[jax.ShapeDtypeStruct((N, _D), jnp.float32),
                   jax.ShapeDtypeStruct((N, _D), jnp.float32)],
        compiler_params=pltpu.CompilerParams(dimension_semantics=("parallel",)),
    )(A0, W, add)


def _a1_body(a_ref, c_ref, e_ref, p_ref, o_ref):
    o_ref[...] = _dott(a_ref[...] + c_ref[...], e_ref[...]) + p_ref[...]


def _a1_pass(A1, coA1, edge, p, bn):
    M, N = A1.shape
    return pl.pallas_call(
        _a1_body,
        grid=(N // bn,),
        in_specs=[pl.BlockSpec((M, bn), lambda j: (0, j)),
                  pl.BlockSpec((M, bn), lambda j: (0, j)),
                  pl.BlockSpec((M, _D), lambda j: (0, 0)),
                  pl.BlockSpec((bn, _D), lambda j: (j, 0))],
        out_specs=pl.BlockSpec((bn, _D), lambda j: (j, 0)),
        out_shape=jax.ShapeDtypeStruct((N, _D), jnp.float32),
        compiler_params=pltpu.CompilerParams(dimension_semantics=("parallel",)),
    )(A1, coA1, edge, p)


def _pool_body(idx_ref, h0_ref, o_ref):
    idx = idx_ref[0, :]
    onehot = (jax.lax.broadcasted_iota(jnp.int32, (_G, _N0), 0)
              == idx[None, :]).astype(jnp.float32)
    s = _dot(onehot, h0_ref[...])
    cnt = jnp.sum(onehot, axis=1, keepdims=True)
    o_ref[...] = s / jnp.maximum(cnt, 1.0)


def _pool(batch_idx2d, h0):
    return pl.pallas_call(
        _pool_body,
        in_specs=[pl.BlockSpec((1, _N0), lambda: (0, 0)),
                  pl.BlockSpec((_N0, _D), lambda: (0, 0))],
        out_specs=pl.BlockSpec((_G, _D), lambda: (0, 0)),
        out_shape=jax.ShapeDtypeStruct((_G, _D), jnp.float32),
    )(batch_idx2d, h0)


def kernel(x, edge_attr, sse_attr, B0, B1, A0, A1, coA1, batch_idx):
    t1h = _mm(B1, sse_attr * 0.5, bm=1024)
    y2 = _mm(B0, jnp.concatenate([t1h, edge_attr], axis=1), bm=256)
    t2, b0e = y2[:, :_D], y2[:, _D:]
    h0, u = _a0_pass(A0, jnp.concatenate([x, t2], axis=1), t2 + b0e, bn=256)
    y4 = _mmt(B0, jnp.concatenate([t2 + x, 0.5 * (u + x)], axis=1), bn=256)
    p, q = y4[:, :_D], y4[:, _D:]
    h1 = _a1_pass(A1, coA1, edge_attr, p, bn=256)
    h2 = _mmt(B1, q + edge_attr, bn=128)
    graph_emb = _pool(batch_idx.reshape(1, _N0).astype(jnp.int32), h0)
    return h0, h1, h2, graph_emb


# speedup vs baseline: 1.7380x; 1.1267x over previous
import jax
import jax.numpy as jnp
from jax.experimental import pallas as pl
from jax.experimental.pallas import tpu as pltpu

_N0, _N1, _N2, _D, _G = 2048, 4096, 512, 32, 8


def _dot(a, b):
    return jax.lax.dot_general(a, b, (((1,), (0,)), ((), ())),
                               preferred_element_type=jnp.float32)


def _dott(a, b):
    return jax.lax.dot_general(a, b, (((0,), (0,)), ((), ())),
                               preferred_element_type=jnp.float32)


def _k1_body(b1_ref, s_ref, o_ref):
    o_ref[...] = _dot(b1_ref[...], s_ref[...] * 0.5)


def _k1(B1, sse, bm):
    return pl.pallas_call(
        _k1_body,
        grid=(_N1 // bm,),
        in_specs=[pl.BlockSpec((bm, _N2), lambda i: (i, 0)),
                  pl.BlockSpec((_N2, _D), lambda i: (0, 0))],
        out_specs=pl.BlockSpec((bm, _D), lambda i: (i, 0)),
        out_shape=jax.ShapeDtypeStruct((_N1, _D), jnp.float32),
        compiler_params=pltpu.CompilerParams(dimension_semantics=("parallel",)),
    )(B1, sse)


def _k2_body(b0_ref, a0_ref, t1_ref, e_ref, x_ref, idx_ref,
             t2_ref, h0_ref, u_ref, ge_ref):
    i = pl.program_id(0)
    nsteps = pl.num_programs(0)
    bm = b0_ref.shape[0]

    t2b = _dot(b0_ref[...], t1_ref[...])
    sb = t2b + _dot(b0_ref[...], e_ref[...])
    t2_ref[...] = t2b

    @pl.when(i == 0)
    def _init():
        h0_ref[...] = jnp.zeros_like(h0_ref)
        u_ref[...] = jnp.zeros_like(u_ref)

    h0_ref[...] += _dott(a0_ref[...], x_ref[...])
    u_ref[...] += _dott(a0_ref[...], t2b)
    h0_ref[pl.ds(i * bm, bm), :] += sb

    @pl.when(i == nsteps - 1)
    def _pool():
        idx = idx_ref[0, :]
        onehot = (jax.lax.broadcasted_iota(jnp.int32, (_G, _N0), 0)
                  == idx[None, :]).astype(jnp.float32)
        s = _dot(onehot, h0_ref[...])
        cnt = jnp.sum(onehot, axis=1, keepdims=True)
        ge_ref[...] = s / jnp.maximum(cnt, 1.0)


def _k2(B0, A0, t1, edge, x, idx2d, bm):
    return pl.pallas_call(
        _k2_body,
        grid=(_N0 // bm,),
        in_specs=[pl.BlockSpec((bm, _N1), lambda i: (i, 0)),
                  pl.BlockSpec((bm, _N0), lambda i: (i, 0)),
                  pl.BlockSpec((_N1, _D), lambda i: (0, 0)),
                  pl.BlockSpec((_N1, _D), lambda i: (0, 0)),
                  pl.BlockSpec((bm, _D), lambda i: (i, 0)),
                  pl.BlockSpec((1, _N0), lambda i: (0, 0))],
        out_specs=[pl.BlockSpec((bm, _D), lambda i: (i, 0)),
                   pl.BlockSpec((_N0, _D), lambda i: (0, 0)),
                   pl.BlockSpec((_N0, _D), lambda i: (0, 0)),
                   pl.BlockSpec((_G, _D), lambda i: (0, 0))],
        out_shape=[jax.ShapeDtypeStruct((_N0, _D), jnp.float32),
                   jax.ShapeDtypeStruct((_N0, _D), jnp.float32),
                   jax.ShapeDtypeStruct((_N0, _D), jnp.float32),
                   jax.ShapeDtypeStruct((_G, _D), jnp.float32)],
        compiler_params=pltpu.CompilerParams(dimension_semantics=("arbitrary",)),
    )(B0, A0, t1, edge, x, idx2d)


def _k3_body(b0_ref, t2_ref, x_ref, u_ref, p_ref, q_ref):
    i = pl.program_id(0)

    @pl.when(i == 0)
    def _init():
        p_ref[...] = jnp.zeros_like(p_ref)
        q_ref[...] = jnp.zeros_like(q_ref)

    xb = x_ref[...]
    p_ref[...] += _dott(b0_ref[...], t2_ref[...] + xb)
    q_ref[...] += _dott(b0_ref[...], (u_ref[...] + xb) * 0.5)


def _k3(B0, t2, x, u, bm):
    return pl.pallas_call(
        _k3_body,
        grid=(_N0 // bm,),
        in_specs=[pl.BlockSpec((bm, _N1), lambda i: (i, 0)),
                  pl.BlockSpec((bm, _D), lambda i: (i, 0)),
                  pl.BlockSpec((bm, _D), lambda i: (i, 0)),
                  pl.BlockSpec((bm, _D), lambda i: (i, 0))],
        out_specs=[pl.BlockSpec((_N1, _D), lambda i: (0, 0)),
                   pl.BlockSpec((_N1, _D), lambda i: (0, 0))],
        out_shape=[jax.ShapeDtypeStruct((_N1, _D), jnp.float32),
                   jax.ShapeDtypeStruct((_N1, _D), jnp.float32)],
        compiler_params=pltpu.CompilerParams(dimension_semantics=("arbitrary",)),
    )(B0, t2, x, u)


def _k4_body(a1_ref, co_ref, b1_ref, e_ref, q_ref, p_ref, h1_ref, h2_ref):
    i = pl.program_id(0)

    @pl.when(i == 0)
    def _init():
        h1_ref[...] = p_ref[...]
        h2_ref[...] = jnp.zeros_like(h2_ref)

    eb = e_ref[...]
    h1_ref[...] += _dott(a1_ref[...] + co_ref[...], eb)
    h2_ref[...] += _dott(b1_ref[...], q_ref[...] + eb)


def _k4(A1, coA1, B1, edge, q, p, bm):
    return pl.pallas_call(
        _k4_body,
        grid=(_N1 // bm,),
        in_specs=[pl.BlockSpec((bm, _N1), lambda i: (i, 0)),
                  pl.BlockSpec((bm, _N1), lambda i: (i, 0)),
                  pl.BlockSpec((bm, _N2), lambda i: (i, 0)),
                  pl.BlockSpec((bm, _D), lambda i: (i, 0)),
                  pl.BlockSpec((bm, _D), lambda i: (i, 0)),
                  pl.BlockSpec((_N1, _D), lambda i: (0, 0))],
        out_specs=[pl.BlockSpec((_N1, _D), lambda i: (0, 0)),
                   pl.BlockSpec((_N2, _D), lambda i: (0, 0))],
        out_shape=[jax.ShapeDtypeStruct((_N1, _D), jnp.float32),
                   jax.ShapeDtypeStruct((_N2, _D), jnp.float32)],
        compiler_params=pltpu.CompilerParams(dimension_semantics=("arbitrary",)),
    )(A1, coA1, B1, edge, q, p)


def kernel(x, edge_attr, sse_attr, B0, B1, A0, A1, coA1, batch_idx):
    idx2d = batch_idx.reshape(1, _N0).astype(jnp.int32)
    t1 = _k1(B1, sse_attr, bm=1024)
    t2, h0, u, graph_emb = _k2(B0, A0, t1, edge_attr, x, idx2d, bm=256)
    p, q = _k3(B0, t2, x, u, bm=256)
    h1, h2 = _k4(A1, coA1, B1, edge_attr, q, p, bm=256)
    return h0, h1, h2, graph_emb


# speedup vs baseline: 1.8034x; 1.0376x over previous
import jax
import jax.numpy as jnp
from jax.experimental import pallas as pl
from jax.experimental.pallas import tpu as pltpu

_N0, _N1, _N2, _D, _G = 2048, 4096, 512, 32, 8


def _dot(a, b):
    return jax.lax.dot_general(a, b, (((1,), (0,)), ((), ())),
                               preferred_element_type=jnp.float32)


def _dott(a, b):
    return jax.lax.dot_general(a, b, (((0,), (0,)), ((), ())),
                               preferred_element_type=jnp.float32)


def _k1_body(b1_ref, s_ref, e_ref, o_ref):
    o_ref[:, :_D] = _dot(b1_ref[...], s_ref[...] * 0.5)
    o_ref[:, _D:] = e_ref[...]


def _k1(B1, sse, edge, bm):
    return pl.pallas_call(
        _k1_body,
        grid=(_N1 // bm,),
        in_specs=[pl.BlockSpec((bm, _N2), lambda i: (i, 0)),
                  pl.BlockSpec((_N2, _D), lambda i: (0, 0)),
                  pl.BlockSpec((bm, _D), lambda i: (i, 0))],
        out_specs=pl.BlockSpec((bm, 2 * _D), lambda i: (i, 0)),
        out_shape=jax.ShapeDtypeStruct((_N1, 2 * _D), jnp.float32),
        compiler_params=pltpu.CompilerParams(dimension_semantics=("parallel",)),
    )(B1, sse, edge)


def _k2_body(b0_ref, a0_ref, rhs_ref, x_ref, idx_ref,
             w1_ref, h0_ref, u_ref, ge_ref):
    i = pl.program_id(0)
    nsteps = pl.num_programs(0)
    bm = b0_ref.shape[0]

    y = _dot(b0_ref[...], rhs_ref[...])
    t2b = y[:, :_D]
    xb = x_ref[...]
    w1_ref[...] = t2b + xb

    z = _dott(a0_ref[...], jnp.concatenate([xb, t2b], axis=1))

    @pl.when(i == 0)
    def _init():
        h0_ref[...] = jnp.zeros_like(h0_ref)
        u_ref[...] = jnp.zeros_like(u_ref)

    h0_ref[...] += z[:, :_D]
    u_ref[...] += z[:, _D:]
    h0_ref[pl.ds(i * bm, bm), :] += t2b + y[:, _D:]

    @pl.when(i == nsteps - 1)
    def _pool():
        idx = idx_ref[0, :]
        onehot = (jax.lax.broadcasted_iota(jnp.int32, (_G, _N0), 0)
                  == idx[None, :]).astype(jnp.float32)
        s = _dot(onehot, h0_ref[...])
        cnt = jnp.sum(onehot, axis=1, keepdims=True)
        ge_ref[...] = s / jnp.maximum(cnt, 1.0)


def _k2(B0, A0, rhs, x, idx2d, bm):
    return pl.pallas_call(
        _k2_body,
        grid=(_N0 // bm,),
        in_specs=[pl.BlockSpec((bm, _N1), lambda i: (i, 0)),
                  pl.BlockSpec((bm, _N0), lambda i: (i, 0)),
                  pl.BlockSpec((_N1, 2 * _D), lambda i: (0, 0)),
                  pl.BlockSpec((bm, _D), lambda i: (i, 0)),
                  pl.BlockSpec((1, _N0), lambda i: (0, 0))],
        out_specs=[pl.BlockSpec((bm, _D), lambda i: (i, 0)),
                   pl.BlockSpec((_N0, _D), lambda i: (0, 0)),
                   pl.BlockSpec((_N0, _D), lambda i: (0, 0)),
                   pl.BlockSpec((_G, _D), lambda i: (0, 0))],
        out_shape=[jax.ShapeDtypeStruct((_N0, _D), jnp.float32),
                   jax.ShapeDtypeStruct((_N0, _D), jnp.float32),
                   jax.ShapeDtypeStruct((_N0, _D), jnp.float32),
                   jax.ShapeDtypeStruct((_G, _D), jnp.float32)],
        compiler_params=pltpu.CompilerParams(dimension_semantics=("arbitrary",)),
    )(B0, A0, rhs, x, idx2d)


def _k3_body(b0_ref, w1_ref, x_ref, u_ref, p_ref, q_ref):
    i = pl.program_id(0)

    @pl.when(i == 0)
    def _init():
        p_ref[...] = jnp.zeros_like(p_ref)
        q_ref[...] = jnp.zeros_like(q_ref)

    w2 = (u_ref[...] + x_ref[...]) * 0.5
    y = _dott(b0_ref[...], jnp.concatenate([w1_ref[...], w2], axis=1))
    p_ref[...] += y[:, :_D]
    q_ref[...] += y[:, _D:]


def _k3(B0, w1, x, u, bm):
    return pl.pallas_call(
        _k3_body,
        grid=(_N0 // bm,),
        in_specs=[pl.BlockSpec((bm, _N1), lambda i: (i, 0)),
                  pl.BlockSpec((bm, _D), lambda i: (i, 0)),
                  pl.BlockSpec((bm, _D), lambda i: (i, 0)),
                  pl.BlockSpec((bm, _D), lambda i: (i, 0))],
        out_specs=[pl.BlockSpec((_N1, _D), lambda i: (0, 0)),
                   pl.BlockSpec((_N1, _D), lambda i: (0, 0))],
        out_shape=[jax.ShapeDtypeStruct((_N1, _D), jnp.float32),
                   jax.ShapeDtypeStruct((_N1, _D), jnp.float32)],
        compiler_params=pltpu.CompilerParams(dimension_semantics=("arbitrary",)),
    )(B0, w1, x, u)


def _k4_body(a1_ref, co_ref, b1_ref, e_ref, q_ref, p_ref, h1_ref, h2_ref):
    i = pl.program_id(0)

    @pl.when(i == 0)
    def _init():
        h1_ref[...] = p_ref[...]
        h2_ref[...] = jnp.zeros_like(h2_ref)

    eb = e_ref[...]
    h1_ref[...] += _dott(a1_ref[...] + co_ref[...], eb)
    h2_ref[...] += _dott(b1_ref[...], q_ref[...] + eb)


def _k4(A1, coA1, B1, edge, q, p, bm):
    return pl.pallas_call(
        _k4_body,
        grid=(_N1 // bm,),
        in_specs=[pl.BlockSpec((bm, _N1), lambda i: (i, 0)),
                  pl.BlockSpec((bm, _N1), lambda i: (i, 0)),
                  pl.BlockSpec((bm, _N2), lambda i: (i, 0)),
                  pl.BlockSpec((bm, _D), lambda i: (i, 0)),
                  pl.BlockSpec((bm, _D), lambda i: (i, 0)),
                  pl.BlockSpec((_N1, _D), lambda i: (0, 0))],
        out_specs=[pl.BlockSpec((_N1, _D), lambda i: (0, 0)),
                   pl.BlockSpec((_N2, _D), lambda i: (0, 0))],
        out_shape=[jax.ShapeDtypeStruct((_N1, _D), jnp.float32),
                   jax.ShapeDtypeStruct((_N2, _D), jnp.float32)],
        compiler_params=pltpu.CompilerParams(dimension_semantics=("arbitrary",)),
    )(A1, coA1, B1, edge, q, p)


def kernel(x, edge_attr, sse_attr, B0, B1, A0, A1, coA1, batch_idx):
    idx2d = batch_idx.reshape(1, _N0).astype(jnp.int32)
    rhs = _k1(B1, sse_attr, edge_attr, bm=1024)
    w1, h0, u, graph_emb = _k2(B0, A0, rhs, x, idx2d, bm=512)
    p, q = _k3(B0, w1, x, u, bm=512)
    h1, h2 = _k4(A1, coA1, B1, edge_attr, q, p, bm=512)
    return h0, h1, h2, graph_emb


# speedup vs baseline: 2.0998x; 1.1644x over previous
import jax
import jax.numpy as jnp
from jax.experimental import pallas as pl
from jax.experimental.pallas import tpu as pltpu

_N0, _N1, _N2, _D, _G = 2048, 4096, 512, 32, 8
_BM = 128
_S1 = _N0 // _BM
_S = _N1 // _BM


def _dot(a, b):
    return jax.lax.dot_general(a, b, (((1,), (0,)), ((), ())),
                               preferred_element_type=jnp.float32)


def _dott(a, b):
    return jax.lax.dot_general(a, b, (((0,), (0,)), ((), ())),
                               preferred_element_type=jnp.float32)


def _body(a1_ref, co_ref, b0_ref, a0_ref, b1_ref, e_ref, x_ref, s_ref,
          idx_ref, h0_ref, h1_ref, h2_ref, ge_ref,
          rhs_scr, b0h_scr, u_scr, p_scr, q_scr):
    i = pl.program_id(0)

    @pl.when(i == 0)
    def _init():
        h0_ref[...] = jnp.zeros_like(h0_ref)
        h1_ref[...] = jnp.zeros_like(h1_ref)
        u_scr[...] = jnp.zeros_like(u_scr)
        p_scr[...] = jnp.zeros_like(p_scr)
        q_scr[...] = jnp.zeros_like(q_scr)
        rhs_scr[:, :_D] = _dot(b1_ref[...], s_ref[...] * 0.5)
        rhs_scr[:, _D:] = e_ref[...]

    eb = e_ref[pl.ds(i * _BM, _BM), :]
    h1_ref[...] += _dott(a1_ref[...] + co_ref[...], eb)

    @pl.when(i < _S1)
    def _phase1():
        b0b = b0_ref[...]
        y = _dot(b0b, rhs_scr[...])
        t2b = y[:, :_D]
        xb = x_ref[pl.ds(i * _BM, _BM), :]
        z = _dott(a0_ref[...], jnp.concatenate([xb, t2b], axis=1))
        h0_ref[...] += z[:, :_D]
        u_scr[...] += z[:, _D:]
        h0_ref[pl.ds(i * _BM, _BM), :] += t2b + y[:, _D:]
        p_scr[...] += _dott(b0b, t2b + xb)
        b0h_scr[pl.ds(i * _BM, _BM), :] = b0b.astype(jnp.bfloat16)

    @pl.when(i >= _S1)
    def _phase2():
        j = i - _S1
        xb = x_ref[pl.ds(j * _BM, _BM), :]
        ub = u_scr[pl.ds(j * _BM, _BM), :]
        w2 = ((ub + xb) * 0.5).astype(jnp.bfloat16)
        q_scr[...] += _dott(b0h_scr[pl.ds(j * _BM, _BM), :], w2)

    @pl.when(i == _S - 1)
    def _fini():
        h1_ref[...] += p_scr[...]
        h2_ref[...] = _dott(b1_ref[...], q_scr[...] + e_ref[...])
        idx = idx_ref[0, :]
        onehot = (jax.lax.broadcasted_iota(jnp.int32, (_G, _N0), 0)
                  == idx[None, :]).astype(jnp.float32)
        s = _dot(onehot, h0_ref[...])
        cnt = jnp.sum(onehot, axis=1, keepdims=True)
        ge_ref[...] = s / jnp.maximum(cnt, 1.0)


def kernel(x, edge_attr, sse_attr, B0, B1, A0, A1, coA1, batch_idx):
    idx2d = batch_idx.reshape(1, _N0).astype(jnp.int32)
    h0, h1, h2, graph_emb = pl.pallas_call(
        _body,
        grid=(_S,),
        in_specs=[
            pl.BlockSpec((_BM, _N1), lambda i: (i, 0)),
            pl.BlockSpec((_BM, _N1), lambda i: (i, 0)),
            pl.BlockSpec((_BM, _N1), lambda i: (jnp.minimum(i, _S1 - 1), 0)),
            pl.BlockSpec((_BM, _N0), lambda i: (jnp.minimum(i, _S1 - 1), 0)),
            pl.BlockSpec((_N1, _N2), lambda i: (0, 0)),
            pl.BlockSpec((_N1, _D), lambda i: (0, 0)),
            pl.BlockSpec((_N0, _D), lambda i: (0, 0)),
            pl.BlockSpec((_N2, _D), lambda i: (0, 0)),
            pl.BlockSpec((1, _N0), lambda i: (0, 0)),
        ],
        out_specs=[
            pl.BlockSpec((_N0, _D), lambda i: (0, 0)),
            pl.BlockSpec((_N1, _D), lambda i: (0, 0)),
            pl.BlockSpec((_N2, _D), lambda i: (0, 0)),
            pl.BlockSpec((_G, _D), lambda i: (0, 0)),
        ],
        out_shape=[
            jax.ShapeDtypeStruct((_N0, _D), jnp.float32),
            jax.ShapeDtypeStruct((_N1, _D), jnp.float32),
            jax.ShapeDtypeStruct((_N2, _D), jnp.float32),
            jax.ShapeDtypeStruct((_G, _D), jnp.float32),
        ],
        scratch_shapes=[
            pltpu.VMEM((_N1, 2 * _D), jnp.float32),
            pltpu.VMEM((_N0, _N1), jnp.bfloat16),
            pltpu.VMEM((_N0, _D), jnp.float32),
            pltpu.VMEM((_N1, _D), jnp.float32),
            pltpu.VMEM((_N1, _D), jnp.float32),
        ],
        compiler_params=pltpu.CompilerParams(dimension_semantics=("arbitrary",)),
    )(A1, coA1, B0, A0, B1, edge_attr, x, sse_attr, idx2d)
    return h0, h1, h2, graph_emb


# speedup vs baseline: 2.3437x; 1.1162x over previous
import jax
import jax.numpy as jnp
from jax.experimental import pallas as pl
from jax.experimental.pallas import tpu as pltpu

_N0, _N1, _N2, _D, _G = 2048, 4096, 512, 32, 8
_BM0 = 256
_S0 = _N0 // _BM0
_BMQ = 128
_SQ = _N0 // _BMQ
_BM1 = 128
_S = _N1 // _BM1

_BF = jnp.bfloat16


def _dot(a, b):
    return jax.lax.dot_general(a, b, (((1,), (0,)), ((), ())),
                               preferred_element_type=jnp.float32)


def _body(a1_ref, co_ref, b0_ref, a0_ref, b1_ref, e_ref, x_ref, s_ref,
          idx_ref, h0_ref, h1_ref, h2_ref, ge_ref,
          rhs_scr, b0h_scr, b1h_scr, xt_scr, w1t_scr,
          h0t_scr, h1t_scr, ut_scr, pt_scr, qt_scr):
    i = pl.program_id(0)

    @pl.when(i == 0)
    def _init():
        h0t_scr[...] = jnp.zeros_like(h0t_scr)
        h1t_scr[...] = jnp.zeros_like(h1t_scr)
        ut_scr[...] = jnp.zeros_like(ut_scr)
        pt_scr[...] = jnp.zeros_like(pt_scr)
        qt_scr[...] = jnp.zeros_like(qt_scr)
        b1h = b1_ref[...].astype(_BF)
        b1h_scr[...] = b1h
        xt_scr[...] = x_ref[...].T
        t1 = _dot(b1h, (s_ref[...] * 0.5).astype(_BF))
        rhs_scr[:, :_D] = t1.astype(_BF)
        rhs_scr[:, _D:] = e_ref[...].astype(_BF)

    ebt = e_ref[pl.ds(i * _BM1, _BM1), :].T.astype(_BF)
    h1t_scr[...] += _dot(ebt, (a1_ref[...] + co_ref[...]).astype(_BF))

    @pl.when(i < _S0)
    def _phase1():
        b0h = b0_ref[...].astype(_BF)
        b0h_scr[pl.ds(i * _BM0, _BM0), :] = b0h
        y = _dot(b0h, rhs_scr[...])
        yt = y.T
        t2bt = yt[:_D, :]
        xbt = xt_scr[:, pl.ds(i * _BM0, _BM0)]
        w1t_scr[:, pl.ds(i * _BM0, _BM0)] = t2bt + xbt
        lhs = jnp.concatenate([xbt, t2bt], axis=0).astype(_BF)
        z = _dot(lhs, a0_ref[...].astype(_BF))
        h0t_scr[...] += z[:_D, :]
        ut_scr[...] += z[_D:, :]
        h0t_scr[:, pl.ds(i * _BM0, _BM0)] += t2bt + yt[_D:, :]

    @pl.when(jnp.logical_and(i >= _S0, i < _S0 + _SQ))
    def _phase2():
        j = i - _S0
        w1tb = w1t_scr[:, pl.ds(j * _BMQ, _BMQ)]
        xbt = xt_scr[:, pl.ds(j * _BMQ, _BMQ)]
        ubt = ut_scr[:, pl.ds(j * _BMQ, _BMQ)]
        w2tb = (ubt + xbt) * 0.5
        lhs = jnp.concatenate([w1tb, w2tb], axis=0).astype(_BF)
        pq = _dot(lhs, b0h_scr[pl.ds(j * _BMQ, _BMQ), :])
        pt_scr[...] += pq[:_D, :]
        qt_scr[...] += pq[_D:, :]

    @pl.when(i == _S - 1)
    def _fini():
        h1t = h1t_scr[...] + pt_scr[...]
        h1_ref[...] = h1t.T
        h0 = h0t_scr[...].T
        h0_ref[...] = h0
        h2t = _dot((qt_scr[...] + e_ref[...].T).astype(_BF), b1h_scr[...])
        h2_ref[...] = h2t.T
        idx = idx_ref[0, :]
        onehot = (jax.lax.broadcasted_iota(jnp.int32, (_G, _N0), 0)
                  == idx[None, :]).astype(jnp.float32)
        s = _dot(onehot, h0)
        cnt = jnp.sum(onehot, axis=1, keepdims=True)
        ge_ref[...] = s / jnp.maximum(cnt, 1.0)


def kernel(x, edge_attr, sse_attr, B0, B1, A0, A1, coA1, batch_idx):
    idx2d = batch_idx.reshape(1, _N0).astype(jnp.int32)
    h0, h1, h2, graph_emb = pl.pallas_call(
        _body,
        grid=(_S,),
        in_specs=[
            pl.BlockSpec((_BM1, _N1), lambda i: (i, 0)),
            pl.BlockSpec((_BM1, _N1), lambda i: (i, 0)),
            pl.BlockSpec((_BM0, _N1), lambda i: (jnp.minimum(i, _S0 - 1), 0)),
            pl.BlockSpec((_BM0, _N0), lambda i: (jnp.minimum(i, _S0 - 1), 0)),
            pl.BlockSpec((_N1, _N2), lambda i: (0, 0)),
            pl.BlockSpec((_N1, _D), lambda i: (0, 0)),
            pl.BlockSpec((_N0, _D), lambda i: (0, 0)),
            pl.BlockSpec((_N2, _D), lambda i: (0, 0)),
            pl.BlockSpec((1, _N0), lambda i: (0, 0)),
        ],
        out_specs=[
            pl.BlockSpec((_N0, _D), lambda i: (0, 0)),
            pl.BlockSpec((_N1, _D), lambda i: (0, 0)),
            pl.BlockSpec((_N2, _D), lambda i: (0, 0)),
            pl.BlockSpec((_G, _D), lambda i: (0, 0)),
        ],
        out_shape=[
            jax.ShapeDtypeStruct((_N0, _D), jnp.float32),
            jax.ShapeDtypeStruct((_N1, _D), jnp.float32),
            jax.ShapeDtypeStruct((_N2, _D), jnp.float32),
            jax.ShapeDtypeStruct((_G, _D), jnp.float32),
        ],
        scratch_shapes=[
            pltpu.VMEM((_N1, 2 * _D), _BF),
            pltpu.VMEM((_N0, _N1), _BF),
            pltpu.VMEM((_N1, _N2), _BF),
            pltpu.VMEM((_D, _N0), jnp.float32),
            pltpu.VMEM((_D, _N0), jnp.float32),
            pltpu.VMEM((_D, _N0), jnp.float32),
            pltpu.VMEM((_D, _N1), jnp.float32),
            pltpu.VMEM((_D, _N0), jnp.float32),
            pltpu.VMEM((_D, _N1), jnp.float32),
            pltpu.VMEM((_D, _N1), jnp.float32),
        ],
        compiler_params=pltpu.CompilerParams(dimension_semantics=("arbitrary",)),
    )(A1, coA1, B0, A0, B1, edge_attr, x, sse_attr, idx2d)
    return h0, h1, h2, graph_emb
